# Initial kernel scaffold; baseline (speedup 1.0000x reference)
import jax
import jax.numpy as jnp
from jax.experimental import pallas as pl


def kernel(input, edge_index, edge_weight, W, b):
    raise NotImplementedError("write your pallas kernel here")


# speedup vs baseline: 4.8327x; 4.8327x over previous
import dataclasses
import functools

import jax
import jax.numpy as jnp
from jax import lax
from jax.experimental import pallas as pl
from jax.experimental.pallas import tpu as pltpu
from jax.experimental.pallas import tpu_sc as plsc

NC = 2
NS = 16
NW = NC * NS
LANES = 16
CHUNK = 128


def _sc_aggregate(x, src, dst, w):
    n, d = x.shape
    e = w.shape[0]
    assert e % CHUNK == 0
    nchunks = e // CHUNK
    zrows = 80
    nblocks = n // zrows
    nslice = d // LANES

    mesh = plsc.VectorSubcoreMesh(core_axis_name="c", subcore_axis_name="s")
    cp = pltpu.CompilerParams()
    if "needs_layout_passes" in pltpu.CompilerParams.__dataclass_fields__:
        cp = dataclasses.replace(cp, needs_layout_passes=False)

    @functools.partial(
        pl.kernel,
        mesh=mesh,
        compiler_params=cp,
        out_type=jax.ShapeDtypeStruct((NC, n, d), jnp.float32),
        scratch_types=[
            pltpu.VMEM((CHUNK,), jnp.int32),
            pltpu.VMEM((CHUNK,), jnp.int32),
            pltpu.VMEM((CHUNK,), jnp.float32),
            pltpu.VMEM((CHUNK, d), jnp.float32),
            pltpu.VMEM((zrows, d), jnp.float32),
            pltpu.VMEM_SHARED((n, d), jnp.float32),
            pltpu.SemaphoreType.DMA,
        ],
    )
    def agg(x_hbm, src_hbm, dst_hbm, w_hbm, out_hbm,
            sidx, didx, wv, rows, zbuf, acc, sem):
        cid = lax.axis_index("c")
        sid = lax.axis_index("s")
        wid = sid * NC + cid

        def zrow(r, _):
            for f in range(nslice):
                zbuf.at[r, pl.ds(f * LANES, LANES)][...] = (
                    jnp.zeros((LANES,), jnp.float32))
            return 0
        lax.fori_loop(0, zrows, zrow, 0)
        nb = (nblocks - sid + NS - 1) // NS

        def zcopy(t, _):
            off = pl.multiple_of((sid + NS * t) * zrows, zrows)
            pltpu.sync_copy(zbuf, acc.at[pl.ds(off, zrows)])
            return 0
        lax.fori_loop(0, nb, zcopy, 0)
        plsc.subcore_barrier()

        nj = (nchunks - wid + NW - 1) // NW

        def chunk_body(j, _):
            base = pl.multiple_of((wid + NW * j) * CHUNK, CHUNK)
            pltpu.sync_copy(src_hbm.at[pl.ds(base, CHUNK)], sidx)
            pltpu.sync_copy(dst_hbm.at[pl.ds(base, CHUNK)], didx)
            pltpu.sync_copy(w_hbm.at[pl.ds(base, CHUNK)], wv)
            pltpu.async_copy(x_hbm.at[sidx], rows, sem).wait()

            def edge_body(r, _):
                bidx = jnp.full((LANES,), r, jnp.int32)
                b16 = plsc.load_gather(wv, [bidx])
                for f in range(nslice):
                    sl = rows.at[r, pl.ds(f * LANES, LANES)]
                    sl[...] = sl[...] * b16
                return 0
            lax.fori_loop(0, CHUNK, edge_body, 0)

            pltpu.sync_copy(rows, acc.at[didx], add=True)
            return 0
        lax.fori_loop(0, nj, chunk_body, 0)

        plsc.subcore_barrier()

        def ecopy(t, _):
            off = pl.multiple_of((sid + NS * t) * zrows, zrows)
            pltpu.sync_copy(acc.at[pl.ds(off, zrows)],
                            out_hbm.at[cid].at[pl.ds(off, zrows)])
            return 0
        lax.fori_loop(0, nb, ecopy, 0)

    return agg(x, src, dst, w)


def _tc_combine(partials, W, b2):
    nc, n, d = partials.shape
    dout = W.shape[1]
    bm = 1000

    def mm(p_ref, w_ref, b_ref, o_ref):
        a = p_ref[0] + p_ref[1]
        o_ref[...] = (
            jnp.dot(a, w_ref[...], preferred_element_type=jnp.float32)
            + b_ref[...])

    return pl.pallas_call(
        mm,
        grid=(n // bm,),
        in_specs=[
            pl.BlockSpec((nc, bm, d), lambda i: (0, i, 0)),
            pl.BlockSpec((d, dout), lambda i: (0, 0)),
            pl.BlockSpec((1, dout), lambda i: (0, 0)),
        ],
        out_specs=pl.BlockSpec((bm, dout), lambda i: (i, 0)),
        out_shape=jax.ShapeDtypeStruct((n, dout), jnp.float32),
    )(partials, W, b2)


def kernel(input, edge_index, edge_weight, W, b):
    src = edge_index[1].astype(jnp.int32)
    dst = edge_index[0].astype(jnp.int32)
    ew = edge_weight.astype(jnp.float32)
    partials = _sc_aggregate(input, src, dst, ew)
    return _tc_combine(partials, W, b.reshape(1, -1))
